# Initial kernel scaffold; baseline (speedup 1.0000x reference)
import jax
import jax.numpy as jnp
from jax.experimental import pallas as pl


def kernel(x, tables, W1, b1, W2, b2, W3, b3, Wh, bh):
    raise NotImplementedError("write your pallas kernel here")


# speedup vs baseline: 2.8884x; 2.8884x over previous
import jax
import jax.numpy as jnp
from jax.experimental import pallas as pl
from jax.experimental.pallas import tpu as pltpu
from jax.experimental.pallas import tpu_sc as plsc

_GATHER_WINDOW = 128
_ROW_PAD = 128
_BM = 512


def _sc_gather(tables_flat, flat_idx):
    num_indices = flat_idx.shape[0]
    emb_dim = tables_flat.shape[1]
    info = plsc.get_sparse_core_info()
    nw = info.num_cores * info.num_subcores
    b_per_w = num_indices // nw
    n_chunks = b_per_w // _GATHER_WINDOW
    assert b_per_w % _GATHER_WINDOW == 0 and num_indices % (8 * nw) == 0
    idx3d = flat_idx.reshape(nw, n_chunks, _GATHER_WINDOW)
    mesh = plsc.VectorSubcoreMesh(core_axis_name="c", subcore_axis_name="s")

    @pl.kernel(
        out_type=jax.ShapeDtypeStruct((num_indices, emb_dim), tables_flat.dtype),
        mesh=mesh,
        scratch_types=[
            pltpu.VMEM((n_chunks, _GATHER_WINDOW), jnp.int32),
            pltpu.VMEM((2, _GATHER_WINDOW, emb_dim), tables_flat.dtype),
            pltpu.SemaphoreType.DMA,
            pltpu.SemaphoreType.DMA,
        ],
    )
    def gather_kernel(tbl_hbm, idx_hbm, out_hbm, idx_v, rows_v, sem0, sem1):
        wid = jax.lax.axis_index("s") * info.num_cores + jax.lax.axis_index("c")
        base = wid * b_per_w
        pltpu.sync_copy(idx_hbm.at[wid], idx_v)
        sems = (sem0, sem1)
        gathers = [None, None]
        gathers[0] = pltpu.async_copy(
            tbl_hbm.at[idx_v.at[0]], rows_v.at[0], sems[0]
        )
        for j in range(n_chunks):
            cur = j % 2
            if j + 1 < n_chunks:
                gathers[(j + 1) % 2] = pltpu.async_copy(
                    tbl_hbm.at[idx_v.at[j + 1]], rows_v.at[(j + 1) % 2],
                    sems[(j + 1) % 2],
                )
            gathers[cur].wait()
            pltpu.sync_copy(
                rows_v.at[cur],
                out_hbm.at[pl.ds(base + j * _GATHER_WINDOW, _GATHER_WINDOW)],
            )

    return gather_kernel(tables_flat, idx3d)


def _gelu_exact(v):
    return 0.5 * v * (1.0 + jax.lax.erf(v * 0.7071067811865476))


def _mlp_heads_kernel(
    h_ref, W1_ref, b1_ref, W2_ref, b2_ref, W3_ref, b3_ref, Wh_ref, bh_ref,
    out_ref, z_ref,
):
    c = pl.program_id(1)

    @pl.when(c == 0)
    def _():
        h1 = jnp.dot(
            h_ref[...].astype(jnp.bfloat16), W1_ref[...],
            preferred_element_type=jnp.float32,
        )
        h1 = _gelu_exact(h1 + b1_ref[...])
        h2 = jnp.dot(h1, W2_ref[...], preferred_element_type=jnp.float32)
        h2 = _gelu_exact(h2 + b2_ref[...])
        z = jnp.dot(h2, W3_ref[...], preferred_element_type=jnp.float32)
        z_ref[...] = z + b3_ref[...]

    z_bf = z_ref[...].astype(jnp.bfloat16)
    logits = jnp.dot(z_bf, Wh_ref[c], preferred_element_type=jnp.float32)
    out_ref[0] = logits + bh_ref[c]


def _tc_forward(h, W1, b1, W2, b2, W3, b3, Wh_bf, bh):
    batch, total_emb = h.shape
    n_cols, d_z, vocab = Wh_bf.shape
    d1 = W1.shape[1]
    d2 = W2.shape[1]
    nb = batch // _BM
    grid = (nb, n_cols)
    return pl.pallas_call(
        _mlp_heads_kernel,
        grid=grid,
        in_specs=[
            pl.BlockSpec((_BM, total_emb), lambda i, c: (i, 0)),
            pl.BlockSpec((total_emb, d1), lambda i, c: (0, 0)),
            pl.BlockSpec((1, d1), lambda i, c: (0, 0)),
            pl.BlockSpec((d1, d2), lambda i, c: (0, 0)),
            pl.BlockSpec((1, d2), lambda i, c: (0, 0)),
            pl.BlockSpec((d2, d_z), lambda i, c: (0, 0)),
            pl.BlockSpec((1, d_z), lambda i, c: (0, 0)),
            pl.BlockSpec((n_cols, d_z, vocab), lambda i, c: (0, 0, 0)),
            pl.BlockSpec((n_cols, vocab), lambda i, c: (0, 0)),
        ],
        out_specs=pl.BlockSpec((1, _BM, vocab), lambda i, c: (c, i, 0)),
        out_shape=jax.ShapeDtypeStruct((n_cols, batch, vocab), jnp.float32),
        scratch_shapes=[pltpu.VMEM((_BM, d_z), jnp.float32)],
    )(h, W1, b1, W2, b2, W3, b3, Wh_bf, bh)


def kernel(x, tables, W1, b1, W2, b2, W3, b3, Wh, bh):
    batch, n_cols = x.shape
    vocab, emb = tables.shape[1], tables.shape[2]
    offsets = jnp.arange(n_cols, dtype=jnp.int32) * vocab
    flat_idx = (x.astype(jnp.int32) + offsets[None, :]).reshape(-1)
    tbl_pad = jnp.pad(
        tables.reshape(n_cols * vocab, emb), ((0, 0), (0, _ROW_PAD - emb))
    )
    emb_rows = _sc_gather(tbl_pad, flat_idx)
    h = emb_rows.reshape(batch, n_cols * _ROW_PAD)
    W1_pad = jnp.pad(
        W1.reshape(n_cols, emb, W1.shape[1]),
        ((0, 0), (0, _ROW_PAD - emb), (0, 0)),
    ).reshape(n_cols * _ROW_PAD, W1.shape[1])
    return _tc_forward(
        h,
        W1_pad.astype(jnp.bfloat16), b1.reshape(1, -1),
        W2, b2.reshape(1, -1),
        W3, b3.reshape(1, -1),
        Wh.astype(jnp.bfloat16), bh,
    )


# speedup vs baseline: 3.0050x; 1.0404x over previous
import jax
import jax.numpy as jnp
from jax.experimental import pallas as pl
from jax.experimental.pallas import tpu as pltpu
from jax.experimental.pallas import tpu_sc as plsc

_GATHER_WINDOW = 128
_ROW_PAD = 128
_BM = 512


def _sc_gather(tables_flat, flat_idx):
    num_indices = flat_idx.shape[0]
    emb_dim = tables_flat.shape[1]
    info = plsc.get_sparse_core_info()
    nw = info.num_cores * info.num_subcores
    b_per_w = num_indices // nw
    n_chunks = b_per_w // _GATHER_WINDOW
    assert b_per_w % _GATHER_WINDOW == 0 and num_indices % (8 * nw) == 0
    idx3d = flat_idx.reshape(nw, n_chunks, _GATHER_WINDOW)
    mesh = plsc.VectorSubcoreMesh(core_axis_name="c", subcore_axis_name="s")

    @pl.kernel(
        out_type=jax.ShapeDtypeStruct((num_indices, emb_dim), tables_flat.dtype),
        mesh=mesh,
        scratch_types=[
            pltpu.VMEM((n_chunks, _GATHER_WINDOW), jnp.int32),
            pltpu.VMEM((2, _GATHER_WINDOW, emb_dim), tables_flat.dtype),
            pltpu.SemaphoreType.DMA,
            pltpu.SemaphoreType.DMA,
        ],
    )
    def gather_kernel(tbl_hbm, idx_hbm, out_hbm, idx_v, rows_v, sem0, sem1):
        wid = jax.lax.axis_index("s") * info.num_cores + jax.lax.axis_index("c")
        base = wid * b_per_w
        pltpu.sync_copy(idx_hbm.at[wid], idx_v)
        sems = (sem0, sem1)
        gathers = [None, None]
        gathers[0] = pltpu.async_copy(
            tbl_hbm.at[idx_v.at[0]], rows_v.at[0], sems[0]
        )
        for j in range(n_chunks):
            cur = j % 2
            if j + 1 < n_chunks:
                gathers[(j + 1) % 2] = pltpu.async_copy(
                    tbl_hbm.at[idx_v.at[j + 1]], rows_v.at[(j + 1) % 2],
                    sems[(j + 1) % 2],
                )
            gathers[cur].wait()
            pltpu.sync_copy(
                rows_v.at[cur],
                out_hbm.at[pl.ds(base + j * _GATHER_WINDOW, _GATHER_WINDOW)],
            )

    return gather_kernel(tables_flat, idx3d)


def _gelu_exact(v):
    return 0.5 * v * (1.0 + jax.lax.erf(v * 0.7071067811865476))


def _tc_forward(h4, W1p, b1, W2, b2, W3, b3, Wh, bh):
    ngrp, n_cols, grp, row_pad = h4.shape
    batch = ngrp * grp
    _, d_z, vocab = Wh.shape
    d1 = W1p.shape[2]
    d2 = W2.shape[1]
    nb = batch // _BM
    grp_per_tile = _BM // grp

    def body(h_ref, W1_ref, b1_ref, W2_ref, b2_ref, W3_ref, b3_ref, Wh_ref,
             bh_ref, out_ref, z_ref):
        c = pl.program_id(1)

        @pl.when(c == 0)
        def _():
            hb = h_ref[...]
            acc = None
            for cc in range(n_cols):
                seg = hb[:, cc].reshape(_BM, row_pad).astype(jnp.bfloat16)
                part = jnp.dot(seg, W1_ref[cc],
                               preferred_element_type=jnp.float32)
                acc = part if acc is None else acc + part
            h1 = _gelu_exact(acc + b1_ref[...])
            h2 = jnp.dot(h1, W2_ref[...], preferred_element_type=jnp.float32)
            h2 = _gelu_exact(h2 + b2_ref[...])
            z = jnp.dot(h2, W3_ref[...], preferred_element_type=jnp.float32)
            z_ref[...] = z + b3_ref[...]

        z_bf = z_ref[...].astype(jnp.bfloat16)
        wh_bf = Wh_ref[c].astype(jnp.bfloat16)
        logits = jnp.dot(z_bf, wh_bf, preferred_element_type=jnp.float32)
        out_ref[0] = logits + bh_ref[c]

    return pl.pallas_call(
        body,
        grid=(nb, n_cols),
        in_specs=[
            pl.BlockSpec((grp_per_tile, n_cols, grp, row_pad),
                         lambda i, c: (i, 0, 0, 0)),
            pl.BlockSpec((n_cols, row_pad, d1), lambda i, c: (0, 0, 0)),
            pl.BlockSpec((1, d1), lambda i, c: (0, 0)),
            pl.BlockSpec((d1, d2), lambda i, c: (0, 0)),
            pl.BlockSpec((1, d2), lambda i, c: (0, 0)),
            pl.BlockSpec((d2, d_z), lambda i, c: (0, 0)),
            pl.BlockSpec((1, d_z), lambda i, c: (0, 0)),
            pl.BlockSpec((n_cols, d_z, vocab), lambda i, c: (0, 0, 0)),
            pl.BlockSpec((n_cols, vocab), lambda i, c: (0, 0)),
        ],
        out_specs=pl.BlockSpec((1, _BM, vocab), lambda i, c: (c, i, 0)),
        out_shape=jax.ShapeDtypeStruct((n_cols, batch, vocab), jnp.float32),
        scratch_shapes=[pltpu.VMEM((_BM, d_z), jnp.float32)],
    )(h4, W1p, b1, W2, b2, W3, b3, Wh, bh)


def kernel(x, tables, W1, b1, W2, b2, W3, b3, Wh, bh):
    batch, n_cols = x.shape
    vocab, emb = tables.shape[1], tables.shape[2]
    ngrp = batch // 8
    offsets = jnp.arange(n_cols, dtype=jnp.int32) * vocab
    xi = x.astype(jnp.int32).reshape(ngrp, 8, n_cols).transpose(0, 2, 1)
    flat_idx = (xi + offsets[None, :, None]).reshape(-1)
    tbl_pad = jnp.pad(
        tables.reshape(n_cols * vocab, emb), ((0, 0), (0, _ROW_PAD - emb))
    )
    emb_rows = _sc_gather(tbl_pad, flat_idx)
    h4 = emb_rows.reshape(ngrp, n_cols, 8, _ROW_PAD)
    W1p = jnp.pad(
        W1.reshape(n_cols, emb, W1.shape[1]),
        ((0, 0), (0, _ROW_PAD - emb), (0, 0)),
    ).astype(jnp.bfloat16)
    return _tc_forward(
        h4, W1p, b1.reshape(1, -1),
        W2, b2.reshape(1, -1),
        W3, b3.reshape(1, -1),
        Wh, bh,
    )


# speedup vs baseline: 3.7765x; 1.2567x over previous
import functools

import jax
import jax.numpy as jnp
import numpy as np
from jax.experimental import pallas as pl
from jax.experimental.pallas import tpu as pltpu
from jax.experimental.pallas import tpu_sc as plsc
from jax.experimental.shard_map import shard_map
from jax.sharding import Mesh, PartitionSpec as P

_GATHER_WINDOW = 128
_ROW_PAD = 128
_BM = 512


def _sc_gather(tables_flat, flat_idx):
    num_indices = flat_idx.shape[0]
    emb_dim = tables_flat.shape[1]
    info = plsc.get_sparse_core_info()
    nw = info.num_cores * info.num_subcores
    b_per_w = num_indices // nw
    n_chunks = b_per_w // _GATHER_WINDOW
    assert b_per_w % _GATHER_WINDOW == 0 and num_indices % (8 * nw) == 0
    idx3d = flat_idx.reshape(nw, n_chunks, _GATHER_WINDOW)
    mesh = plsc.VectorSubcoreMesh(core_axis_name="c", subcore_axis_name="s")

    @pl.kernel(
        out_type=jax.ShapeDtypeStruct((num_indices, emb_dim), tables_flat.dtype),
        mesh=mesh,
        scratch_types=[
            pltpu.VMEM((n_chunks, _GATHER_WINDOW), jnp.int32),
            pltpu.VMEM((2, _GATHER_WINDOW, emb_dim), tables_flat.dtype),
            pltpu.SemaphoreType.DMA,
            pltpu.SemaphoreType.DMA,
        ],
    )
    def gather_kernel(tbl_hbm, idx_hbm, out_hbm, idx_v, rows_v, sem0, sem1):
        wid = jax.lax.axis_index("s") * info.num_cores + jax.lax.axis_index("c")
        base = wid * b_per_w
        pltpu.sync_copy(idx_hbm.at[wid], idx_v)
        sems = (sem0, sem1)
        gathers = [None, None]
        gathers[0] = pltpu.async_copy(
            tbl_hbm.at[idx_v.at[0]], rows_v.at[0], sems[0]
        )
        for j in range(n_chunks):
            cur = j % 2
            if j + 1 < n_chunks:
                gathers[(j + 1) % 2] = pltpu.async_copy(
                    tbl_hbm.at[idx_v.at[j + 1]], rows_v.at[(j + 1) % 2],
                    sems[(j + 1) % 2],
                )
            gathers[cur].wait()
            pltpu.sync_copy(
                rows_v.at[cur],
                out_hbm.at[pl.ds(base + j * _GATHER_WINDOW, _GATHER_WINDOW)],
            )

    return gather_kernel(tables_flat, idx3d)


def _gelu_exact(v):
    return 0.5 * v * (1.0 + jax.lax.erf(v * 0.7071067811865476))


def _tc_forward(h4, W1p, b1, W2, b2, W3, b3, Wh, bh):
    ngrp, n_cols, grp, row_pad = h4.shape
    batch = ngrp * grp
    _, d_z, vocab = Wh.shape
    d1 = W1p.shape[2]
    d2 = W2.shape[1]
    nb = batch // _BM
    grp_per_tile = _BM // grp

    def body(h_ref, W1_ref, b1_ref, W2_ref, b2_ref, W3_ref, b3_ref, Wh_ref,
             bh_ref, out_ref, z_ref):
        c = pl.program_id(1)

        @pl.when(c == 0)
        def _():
            hb = h_ref[...]
            acc = None
            for cc in range(n_cols):
                seg = hb[:, cc].reshape(_BM, row_pad).astype(jnp.bfloat16)
                part = jnp.dot(seg, W1_ref[cc],
                               preferred_element_type=jnp.float32)
                acc = part if acc is None else acc + part
            h1 = _gelu_exact(acc + b1_ref[...])
            h2 = jnp.dot(h1, W2_ref[...], preferred_element_type=jnp.float32)
            h2 = _gelu_exact(h2 + b2_ref[...])
            z = jnp.dot(h2, W3_ref[...], preferred_element_type=jnp.float32)
            z_ref[...] = z + b3_ref[...]

        z_bf = z_ref[...].astype(jnp.bfloat16)
        wh_bf = Wh_ref[c].astype(jnp.bfloat16)
        logits = jnp.dot(z_bf, wh_bf, preferred_element_type=jnp.float32)
        out_ref[0] = logits + bh_ref[c]

    return pl.pallas_call(
        body,
        grid=(nb, n_cols),
        in_specs=[
            pl.BlockSpec((grp_per_tile, n_cols, grp, row_pad),
                         lambda i, c: (i, 0, 0, 0)),
            pl.BlockSpec((n_cols, row_pad, d1), lambda i, c: (0, 0, 0)),
            pl.BlockSpec((1, d1), lambda i, c: (0, 0)),
            pl.BlockSpec((d1, d2), lambda i, c: (0, 0)),
            pl.BlockSpec((1, d2), lambda i, c: (0, 0)),
            pl.BlockSpec((d2, d_z), lambda i, c: (0, 0)),
            pl.BlockSpec((1, d_z), lambda i, c: (0, 0)),
            pl.BlockSpec((n_cols, d_z, vocab), lambda i, c: (0, 0, 0)),
            pl.BlockSpec((n_cols, vocab), lambda i, c: (0, 0)),
        ],
        out_specs=pl.BlockSpec((1, _BM, vocab), lambda i, c: (c, i, 0)),
        out_shape=jax.ShapeDtypeStruct((n_cols, batch, vocab), jnp.float32),
        scratch_shapes=[pltpu.VMEM((_BM, d_z), jnp.float32)],
    )(h4, W1p, b1, W2, b2, W3, b3, Wh, bh)


def _forward(x, tables, W1, b1, W2, b2, W3, b3, Wh, bh):
    batch, n_cols = x.shape
    vocab, emb = tables.shape[1], tables.shape[2]
    ngrp = batch // 8
    offsets = jnp.arange(n_cols, dtype=jnp.int32) * vocab
    xi = x.astype(jnp.int32).reshape(ngrp, 8, n_cols).transpose(0, 2, 1)
    flat_idx = (xi + offsets[None, :, None]).reshape(-1)
    tbl_pad = jnp.pad(
        tables.reshape(n_cols * vocab, emb), ((0, 0), (0, _ROW_PAD - emb))
    )
    emb_rows = _sc_gather(tbl_pad, flat_idx)
    h4 = emb_rows.reshape(ngrp, n_cols, 8, _ROW_PAD)
    W1p = jnp.pad(
        W1.reshape(n_cols, emb, W1.shape[1]),
        ((0, 0), (0, _ROW_PAD - emb), (0, 0)),
    ).astype(jnp.bfloat16)
    return _tc_forward(
        h4, W1p, b1.reshape(1, -1),
        W2, b2.reshape(1, -1),
        W3, b3.reshape(1, -1),
        Wh, bh,
    )


def kernel(x, tables, W1, b1, W2, b2, W3, b3, Wh, bh):
    devs = jax.devices()
    nd = 2 if len(devs) >= 2 and x.shape[0] % 2 == 0 else 1
    mesh = Mesh(np.array(devs[:nd]), ("d",))
    rep = P()
    f = shard_map(
        _forward,
        mesh=mesh,
        in_specs=(P("d"), rep, rep, rep, rep, rep, rep, rep, rep, rep),
        out_specs=P(None, "d"),
        check_rep=False,
    )
    return f(x, tables, W1, b1, W2, b2, W3, b3, Wh, bh)


# speedup vs baseline: 4.4801x; 1.1863x over previous
import functools

import jax
import jax.numpy as jnp
import numpy as np
from jax.experimental import pallas as pl
from jax.experimental.pallas import tpu as pltpu
from jax.experimental.pallas import tpu_sc as plsc
from jax.experimental.shard_map import shard_map
from jax.sharding import Mesh, PartitionSpec as P

_GATHER_WINDOW = 128
_ROW_PAD = 128
_BM = 1024


def _sc_gather(tables_flat, flat_idx):
    num_indices = flat_idx.shape[0]
    emb_dim = tables_flat.shape[1]
    info = plsc.get_sparse_core_info()
    nw = info.num_cores * info.num_subcores
    b_per_w = num_indices // nw
    n_chunks = b_per_w // _GATHER_WINDOW
    assert b_per_w % _GATHER_WINDOW == 0 and num_indices % (8 * nw) == 0
    idx3d = flat_idx.reshape(nw, n_chunks, _GATHER_WINDOW)
    mesh = plsc.VectorSubcoreMesh(core_axis_name="c", subcore_axis_name="s")

    @pl.kernel(
        out_type=jax.ShapeDtypeStruct((num_indices, emb_dim), tables_flat.dtype),
        mesh=mesh,
        scratch_types=[
            pltpu.VMEM((n_chunks, _GATHER_WINDOW), jnp.int32),
            pltpu.VMEM((2, _GATHER_WINDOW, emb_dim), tables_flat.dtype),
            pltpu.SemaphoreType.DMA,
            pltpu.SemaphoreType.DMA,
        ],
    )
    def gather_kernel(tbl_hbm, idx_hbm, out_hbm, idx_v, rows_v, sem0, sem1):
        wid = jax.lax.axis_index("s") * info.num_cores + jax.lax.axis_index("c")
        base = wid * b_per_w
        pltpu.sync_copy(idx_hbm.at[wid], idx_v)
        sems = (sem0, sem1)
        gathers = [None, None]
        gathers[0] = pltpu.async_copy(
            tbl_hbm.at[idx_v.at[0]], rows_v.at[0], sems[0]
        )
        for j in range(n_chunks):
            cur = j % 2
            if j + 1 < n_chunks:
                gathers[(j + 1) % 2] = pltpu.async_copy(
                    tbl_hbm.at[idx_v.at[j + 1]], rows_v.at[(j + 1) % 2],
                    sems[(j + 1) % 2],
                )
            gathers[cur].wait()
            pltpu.sync_copy(
                rows_v.at[cur],
                out_hbm.at[pl.ds(base + j * _GATHER_WINDOW, _GATHER_WINDOW)],
            )

    return gather_kernel(tables_flat, idx3d)


def _gelu_exact(v):
    return 0.5 * v * (1.0 + jax.lax.erf(v * 0.7071067811865476))


def _tc_forward(h4, W1p, b1, W2, b2, W3, b3, Wh, bh):
    ngrp, n_cols, grp, row_pad = h4.shape
    batch = ngrp * grp
    _, d_z, vocab = Wh.shape
    d1 = W1p.shape[2]
    d2 = W2.shape[1]
    nb = batch // _BM
    grp_per_tile = _BM // grp

    def body(h_ref, W1_ref, b1_ref, W2_ref, b2_ref, W3_ref, b3_ref, Wh_ref,
             bh_ref, out_ref, z_ref):
        c = pl.program_id(1)

        @pl.when(c == 0)
        def _():
            hb = h_ref[...]
            acc = None
            for cc in range(n_cols):
                seg = hb[:, cc].reshape(_BM, row_pad).astype(jnp.bfloat16)
                part = jnp.dot(seg, W1_ref[cc],
                               preferred_element_type=jnp.float32)
                acc = part if acc is None else acc + part
            h1 = _gelu_exact(acc + b1_ref[...])
            h2 = jnp.dot(h1, W2_ref[...], preferred_element_type=jnp.float32)
            h2 = _gelu_exact(h2 + b2_ref[...])
            z = jnp.dot(h2, W3_ref[...], preferred_element_type=jnp.float32)
            z_ref[...] = z + b3_ref[...]

        z_bf = z_ref[...].astype(jnp.bfloat16)
        wh_bf = Wh_ref[c].astype(jnp.bfloat16)
        logits = jnp.dot(z_bf, wh_bf, preferred_element_type=jnp.float32)
        out_ref[0] = logits + bh_ref[c]

    return pl.pallas_call(
        body,
        grid=(nb, n_cols),
        in_specs=[
            pl.BlockSpec((grp_per_tile, n_cols, grp, row_pad),
                         lambda i, c: (i, 0, 0, 0)),
            pl.BlockSpec((n_cols, row_pad, d1), lambda i, c: (0, 0, 0)),
            pl.BlockSpec((1, d1), lambda i, c: (0, 0)),
            pl.BlockSpec((d1, d2), lambda i, c: (0, 0)),
            pl.BlockSpec((1, d2), lambda i, c: (0, 0)),
            pl.BlockSpec((d2, d_z), lambda i, c: (0, 0)),
            pl.BlockSpec((1, d_z), lambda i, c: (0, 0)),
            pl.BlockSpec((n_cols, d_z, vocab), lambda i, c: (0, 0, 0)),
            pl.BlockSpec((n_cols, vocab), lambda i, c: (0, 0)),
        ],
        out_specs=pl.BlockSpec((1, _BM, vocab), lambda i, c: (c, i, 0)),
        out_shape=jax.ShapeDtypeStruct((n_cols, batch, vocab), jnp.float32),
        scratch_shapes=[pltpu.VMEM((_BM, d_z), jnp.float32)],
    )(h4, W1p, b1, W2, b2, W3, b3, Wh, bh)


def _forward(x, tables, W1, b1, W2, b2, W3, b3, Wh, bh):
    batch, n_cols = x.shape
    vocab, emb = tables.shape[1], tables.shape[2]
    ngrp = batch // 8
    offsets = jnp.arange(n_cols, dtype=jnp.int32) * vocab
    xi = x.astype(jnp.int32).reshape(ngrp, 8, n_cols).transpose(0, 2, 1)
    flat_idx = (xi + offsets[None, :, None]).reshape(-1)
    tbl_pad = jnp.pad(
        tables.reshape(n_cols * vocab, emb), ((0, 0), (0, _ROW_PAD - emb))
    )
    emb_rows = _sc_gather(tbl_pad, flat_idx)
    h4 = emb_rows.reshape(ngrp, n_cols, 8, _ROW_PAD)
    W1p = jnp.pad(
        W1.reshape(n_cols, emb, W1.shape[1]),
        ((0, 0), (0, _ROW_PAD - emb), (0, 0)),
    ).astype(jnp.bfloat16)
    return _tc_forward(
        h4, W1p, b1.reshape(1, -1),
        W2, b2.reshape(1, -1),
        W3, b3.reshape(1, -1),
        Wh, bh,
    )


def kernel(x, tables, W1, b1, W2, b2, W3, b3, Wh, bh):
    devs = jax.devices()
    nd = 2 if len(devs) >= 2 and x.shape[0] % 2 == 0 else 1
    mesh = Mesh(np.array(devs[:nd]), ("d",))
    rep = P()
    f = shard_map(
        _forward,
        mesh=mesh,
        in_specs=(P("d"), rep, rep, rep, rep, rep, rep, rep, rep, rep),
        out_specs=P(None, "d"),
        check_rep=False,
    )
    return f(x, tables, W1, b1, W2, b2, W3, b3, Wh, bh)
